# baseline (speedup 1.0000x reference)
import dataclasses
import functools

import jax
import jax.numpy as jnp
from jax import lax
from jax.experimental import pallas as pl
from jax.experimental.pallas import tpu as pltpu
from jax.experimental.pallas import tpu_sc as plsc

_EPS = 1.1920928955078125e-07

_NC, _NS = 2, 16
_NW = _NC * _NS
_CW = 1024
_ZB = 8192


def _proj_body(c_ref, wl_ref, wm_ref, sl_ref, smp_ref):
    y = jnp.maximum(c_ref[...], 0.0)
    sl_ref[...] = jnp.dot(y, wl_ref[...], preferred_element_type=jnp.float32)
    xb = y.shape[0] // 8
    for j in range(8):
        smp_ref[:, j * 16:(j + 1) * 16] = jnp.dot(
            y[j * xb:(j + 1) * xb], wm_ref[...],
            preferred_element_type=jnp.float32)


def _zp_body(z_ref, g_ref, w_ref, o_ref):
    x = z_ref[...]
    ms = jnp.mean(x * x, axis=1, keepdims=True)
    y = (x * lax.rsqrt(ms + _EPS) * g_ref[...]).astype(jnp.bfloat16)
    w = w_ref[...].astype(jnp.bfloat16)
    xb = y.shape[0] // 8
    for j in range(8):
        o_ref[:, j * 16:(j + 1) * 16] = jnp.dot(
            y[j * xb:(j + 1) * xb], w,
            preferred_element_type=jnp.float32)


def _mlp_body(smg_ref, zpg_ref, sl_ref, b1_ref, b2_ref, b3_ref, o_ref):
    sl = sl_ref[...]
    slt = jnp.concatenate([sl] * 32, axis=1)
    p = smg_ref[...] + zpg_ref[...] + slt
    b1, b2, b3 = b1_ref[...], b2_ref[...], b3_ref[...]
    for g in range(p.shape[1] // 128):
        pg = p[:, g * 128:(g + 1) * 128]
        h = jnp.dot(jnp.maximum(pg, 0.0), b1, preferred_element_type=jnp.float32)
        h = jnp.dot(jnp.maximum(h, 0.0), b2, preferred_element_type=jnp.float32)
        h = jnp.dot(jnp.maximum(h, 0.0), b3, preferred_element_type=jnp.float32)
        o_ref[:, g * 128:(g + 1) * 128] = pg + h


def _sc_params():
    mesh = plsc.VectorSubcoreMesh(core_axis_name="c", subcore_axis_name="s",
                                  num_cores=_NC, num_subcores=_NS)
    cp = pltpu.CompilerParams()
    if "needs_layout_passes" in pltpu.CompilerParams.__dataclass_fields__:
        cp = dataclasses.replace(cp, needs_layout_passes=False)
    if "use_tc_tiling_on_sc" in pltpu.CompilerParams.__dataclass_fields__:
        cp = dataclasses.replace(cp, use_tc_tiling_on_sc=False)
    return mesh, cp


def _sc_gather_sm(ind, smp_lin):
    n = ind.shape[0]
    l_tot = smp_lin.shape[0]
    ca = smp_lin.shape[1]
    per_w = n // _NW
    nch = per_w // _CW
    mesh, cp = _sc_params()

    @functools.partial(
        pl.kernel,
        compiler_params=cp,
        out_type=jax.ShapeDtypeStruct((n, ca), jnp.float32),
        mesh=mesh,
        scratch_types=[
            pltpu.VMEM((_CW,), jnp.int32),
            pltpu.VMEM((_CW, ca), jnp.float32),
            pltpu.SemaphoreType.DMA,
        ],
    )
    def sck(ind_hbm, sm_t, osm_t, idx_v, a_v, sem_a):
        wid = lax.axis_index("s") * _NC + lax.axis_index("c")

        @pl.loop(0, nch)
        def _chunk(ci):
            base = wid * per_w + ci * _CW
            pltpu.sync_copy(ind_hbm.at[pl.ds(base, _CW)], idx_v)

            @pl.loop(0, _CW, step=16)
            def _vec(i):
                jv = idx_v[pl.ds(i, 16)]
                jv = jnp.minimum(jnp.maximum(jv, 0), l_tot - 1)
                idx_v[pl.ds(i, 16)] = ((jv & (l_tot // 8 - 1)) << 3) | (
                    lax.shift_right_logical(jv, 10))

            pltpu.async_copy(sm_t.at[idx_v], a_v, sem_a).wait()
            pltpu.sync_copy(a_v, osm_t.at[pl.ds(base, _CW)])

    return sck(ind, smp_lin)


def _sc_gather_zp(ind, tok, zpp_lin):
    n = ind.shape[0]
    l_tot = tok.shape[0]
    i_tot = 512
    ca = zpp_lin.shape[1]
    per_w = n // _NW
    nch = per_w // _CW
    mesh, cp = _sc_params()

    @functools.partial(
        pl.kernel,
        compiler_params=cp,
        out_type=jax.ShapeDtypeStruct((n, ca), jnp.float32),
        mesh=mesh,
        scratch_types=[
            pltpu.VMEM((l_tot,), jnp.int32),
            pltpu.VMEM((_CW,), jnp.int32),
            pltpu.VMEM((_CW,), jnp.int32),
            pltpu.VMEM((_CW, ca), jnp.float32),
            pltpu.SemaphoreType.DMA,
        ],
    )
    def sck(ind_hbm, tok_hbm, zp_t, ozp_t, tok_v, idx_v, flat_v, b_v, sem_b):
        wid = lax.axis_index("s") * _NC + lax.axis_index("c")
        pltpu.sync_copy(tok_hbm, tok_v)

        @pl.loop(0, nch)
        def _chunk(ci):
            base = wid * per_w + ci * _CW
            pltpu.sync_copy(ind_hbm.at[pl.ds(base, _CW)], idx_v)

            @pl.loop(0, _CW, step=16)
            def _vec(i):
                jv = idx_v[pl.ds(i, 16)]
                jv = jnp.minimum(jnp.maximum(jv, 0), l_tot - 1)
                tv = plsc.load_gather(tok_v, [jv])
                pos = base + i + lax.iota(jnp.int32, 16)
                lv = lax.shift_right_logical(pos, 5)
                qv = plsc.load_gather(tok_v, [lv])
                fv = qv * i_tot + tv
                flat_v[pl.ds(i, 16)] = ((fv & ~(_ZB - 1))
                                        | ((fv & (_ZB // 8 - 1)) << 3)
                                        | (lax.shift_right_logical(fv, 10) & 7))

            pltpu.async_copy(zp_t.at[flat_v], b_v, sem_b).wait()
            pltpu.sync_copy(b_v, ozp_t.at[pl.ds(base, _CW)])

    return sck(ind, tok, zpp_lin)


def kernel(f, indices, C_L, Z_init_II, tok_idx, W_l, W_m, rms_w, W_z, W1, W2, W3):
    d, l, k = indices.shape
    i_tot = Z_init_II.shape[0]
    ct = C_L.shape[-1]
    ca = W_l.shape[0]
    n = d * l * k
    pk = 128 // ca

    c2 = C_L.reshape(l, ct)
    zf = Z_init_II.reshape(i_tot * i_tot, ct)
    ind = indices.reshape(n)

    sl, smp = pl.pallas_call(
        _proj_body,
        out_shape=(jax.ShapeDtypeStruct((l, ca), jnp.float32),
                   jax.ShapeDtypeStruct((l // pk, 128), jnp.float32)),
    )(c2, W_l.T, W_m.T)

    zn = i_tot * i_tot
    zpp = pl.pallas_call(
        _zp_body,
        grid=(zn // _ZB,),
        in_specs=[
            pl.BlockSpec((_ZB, ct), lambda i: (i, 0)),
            pl.BlockSpec((1, ct), lambda i: (0, 0)),
            pl.BlockSpec((ct, ca), lambda i: (0, 0)),
        ],
        out_specs=pl.BlockSpec((_ZB // pk, 128), lambda i: (i, 0)),
        out_shape=jax.ShapeDtypeStruct((zn // pk, 128), jnp.float32),
    )(zf, rms_w.reshape(1, ct), W_z.T)

    smg = _sc_gather_sm(ind, smp.reshape(l, ca))
    zpg = _sc_gather_zp(ind, tok_idx, zpp.reshape(zn, ca))

    eyep = jnp.eye(pk, dtype=jnp.float32)
    b1 = jnp.kron(eyep, W1.T)
    b2 = jnp.kron(eyep, W2.T)
    b3 = jnp.kron(eyep, W3.T)

    rows = 512
    wide = k * ca
    out = pl.pallas_call(
        _mlp_body,
        grid=(l // rows,),
        in_specs=[
            pl.BlockSpec((rows, wide), lambda i: (i, 0)),
            pl.BlockSpec((rows, wide), lambda i: (i, 0)),
            pl.BlockSpec((rows, ca), lambda i: (i, 0)),
            pl.BlockSpec((128, 128), lambda i: (0, 0)),
            pl.BlockSpec((128, 128), lambda i: (0, 0)),
            pl.BlockSpec((128, 128), lambda i: (0, 0)),
        ],
        out_specs=pl.BlockSpec((rows, wide), lambda i: (i, 0)),
        out_shape=jax.ShapeDtypeStruct((l, wide), jnp.float32),
    )(smg.reshape(l, wide), zpg.reshape(l, wide), sl, b1, b2, b3)

    return out.reshape(d, l, k, ca)


# speedup vs baseline: 1.1895x; 1.1895x over previous
import dataclasses
import functools

import jax
import jax.numpy as jnp
from jax import lax
from jax.experimental import pallas as pl
from jax.experimental.pallas import tpu as pltpu
from jax.experimental.pallas import tpu_sc as plsc

_EPS = 1.1920928955078125e-07

_NC, _NS = 2, 16
_NW = _NC * _NS
_CW = 1024
_ZB = 8192


def _proj_body(c_ref, wl_ref, wm_ref, sl_ref, smp_ref):
    y = jnp.maximum(c_ref[...], 0.0)
    sl_ref[...] = jnp.dot(y, wl_ref[...], preferred_element_type=jnp.float32)
    xb = y.shape[0] // 8
    for j in range(8):
        smp_ref[:, j * 16:(j + 1) * 16] = jnp.dot(
            y[j * xb:(j + 1) * xb], wm_ref[...],
            preferred_element_type=jnp.float32)


def _zp_body(z_ref, g_ref, w_ref, o_ref):
    x = z_ref[...]
    ms = jnp.mean(x * x, axis=1, keepdims=True)
    y = x * lax.rsqrt(ms + _EPS) * g_ref[...]
    xb = y.shape[0] // 8
    for j in range(8):
        o_ref[:, j * 16:(j + 1) * 16] = jnp.dot(
            y[j * xb:(j + 1) * xb], w_ref[...],
            preferred_element_type=jnp.float32)


def _mlp_body(smg_ref, zpg_ref, sl_ref, e_ref, b1_ref, b2_ref, b3_ref, o_ref):
    sl = sl_ref[...]
    sl8 = jnp.concatenate([sl] * 8, axis=1)
    slt = jnp.dot(e_ref[...], sl8,
                  preferred_element_type=jnp.float32)
    p = smg_ref[...] + zpg_ref[...] + slt
    h = jnp.dot(jnp.maximum(p, 0.0), b1_ref[...], preferred_element_type=jnp.float32)
    h = jnp.dot(jnp.maximum(h, 0.0), b2_ref[...], preferred_element_type=jnp.float32)
    h = jnp.dot(jnp.maximum(h, 0.0), b3_ref[...], preferred_element_type=jnp.float32)
    r = p + h
    o_ref[...] = r.reshape(r.shape[0] // 4, 512)


def _sc_params():
    mesh = plsc.VectorSubcoreMesh(core_axis_name="c", subcore_axis_name="s",
                                  num_cores=_NC, num_subcores=_NS)
    cp = pltpu.CompilerParams()
    if "needs_layout_passes" in pltpu.CompilerParams.__dataclass_fields__:
        cp = dataclasses.replace(cp, needs_layout_passes=False)
    if "use_tc_tiling_on_sc" in pltpu.CompilerParams.__dataclass_fields__:
        cp = dataclasses.replace(cp, use_tc_tiling_on_sc=False)
    return mesh, cp


def _sc_gather(ind, tok, smp_lin, zpp_lin):
    n = ind.shape[0]
    l_tot = tok.shape[0]
    i_tot = 512
    ca = 16
    per_w = n // _NW
    nch = per_w // _CW
    mesh, cp = _sc_params()

    @functools.partial(
        pl.kernel,
        compiler_params=cp,
        out_type=(jax.ShapeDtypeStruct((n, ca), jnp.float32),
                  jax.ShapeDtypeStruct((n, ca), jnp.float32)),
        mesh=mesh,
        scratch_types=[
            pltpu.VMEM((l_tot,), jnp.int32),
            pltpu.VMEM((_CW,), jnp.int32),
            pltpu.VMEM((_CW,), jnp.int32),
            pltpu.VMEM((_CW, ca), jnp.float32),
            pltpu.VMEM((_CW, ca), jnp.float32),
            pltpu.SemaphoreType.DMA,
            pltpu.SemaphoreType.DMA,
        ],
    )
    def sck(ind_hbm, tok_hbm, sm_t, zp_t, osm_t, ozp_t,
            tok_v, idx_v, flat_v, a_v, b_v, sem_a, sem_b):
        wid = lax.axis_index("s") * _NC + lax.axis_index("c")
        pltpu.sync_copy(tok_hbm, tok_v)

        @pl.loop(0, nch)
        def _chunk(ci):
            base = wid * per_w + ci * _CW
            pltpu.sync_copy(ind_hbm.at[pl.ds(base, _CW)], idx_v)

            @pl.loop(0, _CW, step=16)
            def _vec(i):
                jv = idx_v[pl.ds(i, 16)]
                jv = jnp.minimum(jnp.maximum(jv, 0), l_tot - 1)
                tv = plsc.load_gather(tok_v, [jv])
                pos = base + i + lax.iota(jnp.int32, 16)
                lv = lax.shift_right_logical(pos, 5)
                qv = plsc.load_gather(tok_v, [lv])
                idx_v[pl.ds(i, 16)] = ((jv & (l_tot // 8 - 1)) << 3) | (
                    lax.shift_right_logical(jv, 10))
                fv = qv * i_tot + tv
                flat_v[pl.ds(i, 16)] = ((fv & ~(_ZB - 1))
                                        | ((fv & (_ZB // 8 - 1)) << 3)
                                        | (lax.shift_right_logical(fv, 10) & 7))

            cp_a = pltpu.async_copy(sm_t.at[idx_v], a_v, sem_a)
            cp_b = pltpu.async_copy(zp_t.at[flat_v], b_v, sem_b)
            cp_a.wait()
            cp_b.wait()
            pltpu.sync_copy(a_v, osm_t.at[pl.ds(base, _CW)])
            pltpu.sync_copy(b_v, ozp_t.at[pl.ds(base, _CW)])

    return sck(ind, tok, smp_lin, zpp_lin)


def kernel(f, indices, C_L, Z_init_II, tok_idx, W_l, W_m, rms_w, W_z, W1, W2, W3):
    d, l, k = indices.shape
    i_tot = Z_init_II.shape[0]
    ct = C_L.shape[-1]
    ca = W_l.shape[0]
    n = d * l * k
    pk = 128 // ca

    c2 = C_L.reshape(l, ct)
    zf = Z_init_II.reshape(i_tot * i_tot, ct)
    ind = indices.reshape(n)

    sl, smp = pl.pallas_call(
        _proj_body,
        out_shape=(jax.ShapeDtypeStruct((l, ca), jnp.float32),
                   jax.ShapeDtypeStruct((l // pk, 128), jnp.float32)),
    )(c2, W_l.T, W_m.T)

    zn = i_tot * i_tot
    zpp = pl.pallas_call(
        _zp_body,
        grid=(zn // _ZB,),
        in_specs=[
            pl.BlockSpec((_ZB, ct), lambda i: (i, 0)),
            pl.BlockSpec((1, ct), lambda i: (0, 0)),
            pl.BlockSpec((ct, ca), lambda i: (0, 0)),
        ],
        out_specs=pl.BlockSpec((_ZB // pk, 128), lambda i: (i, 0)),
        out_shape=jax.ShapeDtypeStruct((zn // pk, 128), jnp.float32),
    )(zf, rms_w.reshape(1, ct), W_z.T)

    smg, zpg = _sc_gather(ind, tok_idx,
                          smp.reshape(l, ca),
                          zpp.reshape(zn, ca))
    smg = smg.reshape(n // pk, 128)
    zpg = zpg.reshape(n // pk, 128)

    eyep = jnp.eye(pk, dtype=jnp.float32)
    b1 = jnp.kron(eyep, W1.T)
    b2 = jnp.kron(eyep, W2.T)
    b3 = jnp.kron(eyep, W3.T)

    prows = 2048
    wide = k * ca
    ex = jnp.kron(jnp.eye(prows // 4, dtype=jnp.float32),
                  jnp.ones((4, 1), jnp.float32))
    out = pl.pallas_call(
        _mlp_body,
        grid=(n // pk // prows,),
        in_specs=[
            pl.BlockSpec((prows, 128), lambda i: (i, 0)),
            pl.BlockSpec((prows, 128), lambda i: (i, 0)),
            pl.BlockSpec((prows // 4, ca), lambda i: (i, 0)),
            pl.BlockSpec((prows, prows // 4), lambda i: (0, 0)),
            pl.BlockSpec((128, 128), lambda i: (0, 0)),
            pl.BlockSpec((128, 128), lambda i: (0, 0)),
            pl.BlockSpec((128, 128), lambda i: (0, 0)),
        ],
        out_specs=pl.BlockSpec((prows // 4, wide), lambda i: (i, 0)),
        out_shape=jax.ShapeDtypeStruct((l, wide), jnp.float32),
    )(smg, zpg, sl, ex, b1, b2, b3)

    return out.reshape(d, l, k, ca)
